# Initial kernel scaffold; baseline (speedup 1.0000x reference)
import jax
import jax.numpy as jnp
from jax.experimental import pallas as pl


def kernel(input_idx, type_idx, dpe_idx, E_input, E_type, E_dpe):
    raise NotImplementedError("write your pallas kernel here")


# speedup vs baseline: 8.1313x; 8.1313x over previous
import functools

import jax
import jax.numpy as jnp
from jax import lax
from jax.experimental import pallas as pl
from jax.experimental.pallas import tpu as pltpu
from jax.experimental.pallas import tpu_sc as plsc

D = 128
SEQ = 128
LANES = 16
NVEC = D // LANES


@functools.lru_cache(maxsize=None)
def _build(n_events, vocab_in, vocab_ty, vocab_dp):
  info = plsc.get_sparse_core_info()
  nc, ns = info.num_cores, info.num_subcores
  nw = nc * ns
  assert n_events % nw == 0
  ev_w = n_events // nw

  mesh = plsc.VectorSubcoreMesh(core_axis_name="c", subcore_axis_name="s")

  @functools.partial(
      pl.kernel,
      mesh=mesh,
      out_type=jax.ShapeDtypeStruct((nw, ev_w, D), jnp.float32),
      scratch_types=[
          pltpu.VMEM((ev_w, SEQ), jnp.int32),
          pltpu.VMEM((ev_w, SEQ), jnp.int32),
          pltpu.VMEM((ev_w, SEQ), jnp.int32),
          pltpu.VMEM((3 * SEQ, D), jnp.float32),
          pltpu.VMEM((ev_w, D), jnp.float32),
          pltpu.SemaphoreType.DMA,
      ],
  )
  def encoder(ii_hbm, ti_hbm, di_hbm, tab_i, tab_t, tab_d, out_hbm,
              idx_i, idx_t, idx_d, rows, out_buf, sem):
    wid = lax.axis_index("s") * nc + lax.axis_index("c")

    pltpu.sync_copy(ii_hbm.at[wid], idx_i)
    pltpu.sync_copy(ti_hbm.at[wid], idx_t)
    pltpu.sync_copy(di_hbm.at[wid], idx_d)

    def ev_body(e, carry):
      c1 = pltpu.async_copy(tab_i.at[idx_i.at[e]], rows.at[pl.ds(0, SEQ)], sem)
      c2 = pltpu.async_copy(tab_t.at[idx_t.at[e]], rows.at[pl.ds(SEQ, SEQ)], sem)
      c3 = pltpu.async_copy(tab_d.at[idx_d.at[e]], rows.at[pl.ds(2 * SEQ, SEQ)], sem)
      c1.wait()
      c2.wait()
      c3.wait()

      def red(r, accs):
        return tuple(a + rows[r, pl.ds(j * LANES, LANES)]
                     for j, a in enumerate(accs))

      accs = lax.fori_loop(
          0, 3 * SEQ, red,
          tuple(jnp.zeros((LANES,), jnp.float32) for _ in range(NVEC)),
          unroll=4)
      scale = jnp.float32(1.0 / SEQ)
      for j in range(NVEC):
        out_buf[e, pl.ds(j * LANES, LANES)] = accs[j] * scale
      return carry

    lax.fori_loop(0, ev_w, ev_body, 0)
    pltpu.sync_copy(out_buf, out_hbm.at[wid])

  return encoder


def kernel(input_idx, type_idx, dpe_idx, E_input, E_type, E_dpe):
  b, l, seq = input_idx.shape
  n = b * l
  enc = _build(n, E_input.shape[0], E_type.shape[0], E_dpe.shape[0])
  info = plsc.get_sparse_core_info()
  nw = info.num_cores * info.num_subcores
  out = enc(
      input_idx.reshape(nw, n // nw, seq).astype(jnp.int32),
      type_idx.reshape(nw, n // nw, seq).astype(jnp.int32),
      dpe_idx.reshape(nw, n // nw, seq).astype(jnp.int32),
      E_input, E_type, E_dpe,
  )
  return out.reshape(b, l, D)


# speedup vs baseline: 12.4521x; 1.5314x over previous
import functools

import jax
import jax.numpy as jnp
from jax import lax
from jax.experimental import pallas as pl
from jax.experimental.pallas import tpu as pltpu
from jax.experimental.pallas import tpu_sc as plsc

D = 128
SEQ = 128
LANES = 16
NVEC = D // LANES


@functools.lru_cache(maxsize=None)
def _build(n_events, vocab_in, vocab_ty, vocab_dp):
  info = plsc.get_sparse_core_info()
  nc, ns = info.num_cores, info.num_subcores
  nw = nc * ns
  assert n_events % nw == 0
  ev_w = n_events // nw

  mesh = plsc.VectorSubcoreMesh(core_axis_name="c", subcore_axis_name="s")

  @functools.partial(
      pl.kernel,
      mesh=mesh,
      out_type=jax.ShapeDtypeStruct((nw, ev_w, D), jnp.float32),
      scratch_types=[
          pltpu.VMEM((ev_w, SEQ), jnp.int32),
          pltpu.VMEM((ev_w, SEQ), jnp.int32),
          pltpu.VMEM((ev_w, SEQ), jnp.int32),
          pltpu.VMEM((2 * 3 * SEQ, D), jnp.float32),
          pltpu.VMEM((ev_w, D), jnp.float32),
          pltpu.SemaphoreType.DMA,
          pltpu.SemaphoreType.DMA,
      ],
  )
  def encoder(ii_hbm, ti_hbm, di_hbm, tab_i, tab_t, tab_d, out_hbm,
              idx_i, idx_t, idx_d, rows, out_buf, sem0, sem1):
    wid = lax.axis_index("s") * nc + lax.axis_index("c")

    pltpu.sync_copy(ii_hbm.at[wid], idx_i)
    pltpu.sync_copy(ti_hbm.at[wid], idx_t)
    pltpu.sync_copy(di_hbm.at[wid], idx_d)

    def copies(e, slot_base, sem):
      return (
          pltpu.make_async_copy(
              tab_i.at[idx_i.at[e]], rows.at[pl.ds(slot_base, SEQ)], sem),
          pltpu.make_async_copy(
              tab_t.at[idx_t.at[e]], rows.at[pl.ds(slot_base + SEQ, SEQ)], sem),
          pltpu.make_async_copy(
              tab_d.at[idx_d.at[e]],
              rows.at[pl.ds(slot_base + 2 * SEQ, SEQ)], sem),
      )

    def issue(e, slot_base, sem):
      for c in copies(e, slot_base, sem):
        c.start()

    def wait(e, slot_base, sem):
      for c in copies(e, slot_base, sem):
        c.wait()

    def reduce_into(e, slot_base):
      def red(r, accs):
        return tuple(a + rows[slot_base + r, pl.ds(j * LANES, LANES)]
                     for j, a in enumerate(accs))

      accs = lax.fori_loop(
          0, 3 * SEQ, red,
          tuple(jnp.zeros((LANES,), jnp.float32) for _ in range(NVEC)),
          unroll=4)
      scale = jnp.float32(1.0 / SEQ)
      for j in range(NVEC):
        out_buf[e, pl.ds(j * LANES, LANES)] = accs[j] * scale

    assert ev_w % 2 == 0
    issue(0, 0, sem0)

    def pair_body(k, carry):
      e0 = 2 * k
      issue(e0 + 1, 3 * SEQ, sem1)
      wait(e0, 0, sem0)
      reduce_into(e0, 0)

      @pl.when(e0 + 2 < ev_w)
      def _():
        issue(e0 + 2, 0, sem0)

      wait(e0 + 1, 3 * SEQ, sem1)
      reduce_into(e0 + 1, 3 * SEQ)
      return carry

    lax.fori_loop(0, ev_w // 2, pair_body, 0)
    pltpu.sync_copy(out_buf, out_hbm.at[wid])

  return encoder


def kernel(input_idx, type_idx, dpe_idx, E_input, E_type, E_dpe):
  b, l, seq = input_idx.shape
  n = b * l
  enc = _build(n, E_input.shape[0], E_type.shape[0], E_dpe.shape[0])
  info = plsc.get_sparse_core_info()
  nw = info.num_cores * info.num_subcores
  out = enc(
      input_idx.reshape(nw, n // nw, seq).astype(jnp.int32),
      type_idx.reshape(nw, n // nw, seq).astype(jnp.int32),
      dpe_idx.reshape(nw, n // nw, seq).astype(jnp.int32),
      E_input, E_type, E_dpe,
  )
  return out.reshape(b, l, D)
